# baseline (device time: 334639 ns/iter reference)
import jax
import jax.numpy as jnp
from jax import lax
from jax.experimental import pallas as pl
from jax.experimental.pallas import tpu as pltpu

N_DEV = 4
NT = 512
H = NT // 2


def kernel(x, w_mat):
    m_total, k_loc = x.shape
    k_loc2, n_total = w_mat.shape
    assert k_loc == k_loc2
    x = x.astype(jnp.bfloat16)
    w_mat = w_mat.astype(jnp.bfloat16)
    m_blk = m_total // N_DEV
    T = n_total // NT
    S = T + 3

    def body(x_ref, w0_ref, w1_ref, w2_ref, w3_ref, out_ref,
             comm_r, comm_l, send_r, recv_r, send_l, recv_l,
             credit_r, credit_l):
        s = pl.program_id(0)
        my = lax.axis_index("i")
        left = (my + N_DEV - 1) % N_DEV
        right = (my + 1) % N_DEV

        @pl.when(s == 0)
        def _():
            barrier_sem = pltpu.get_barrier_semaphore()
            for nbr in (left, right):
                pl.semaphore_signal(
                    barrier_sem, inc=1,
                    device_id=(nbr,), device_id_type=pl.DeviceIdType.MESH,
                )
            pl.semaphore_wait(barrier_sem, 2)

        @pl.when(s >= 1)
        def _():
            pl.semaphore_wait(credit_r, 1)
            pl.semaphore_wait(credit_l, 1)

        sp = [(s + 4 - k) % 4 for k in range(4)]
        pp = [(s + 2 - k) % 2 for k in range(4)]

        def part(j, w_ref, lo):
            xb = x_ref[pl.ds(j * m_blk, m_blk), :]
            return jnp.dot(
                xb, w_ref[:, lo:lo + H], preferred_element_type=jnp.float32
            )

        dirs = (
            (comm_r, send_r, recv_r, right, 0,
             ((my + 3) % N_DEV, (my + 2) % N_DEV, (my + 1) % N_DEV, my)),
            (comm_l, send_l, recv_l, left, H,
             ((my + 1) % N_DEV, (my + 2) % N_DEV, (my + 3) % N_DEV, my)),
        )

        def fwd(comm, ssems, rsems, dev, src, dst, spar, ppar, h):
            return pltpu.make_async_remote_copy(
                src_ref=comm.at[spar, src],
                dst_ref=comm.at[spar, dst],
                send_sem=ssems.at[ppar, h],
                recv_sem=rsems.at[ppar, h],
                device_id=(dev,),
                device_id_type=pl.DeviceIdType.MESH,
            )

        @pl.when(s < T)
        def _():
            for comm, ssems, rsems, dev, lo, chunks in dirs:
                d = fwd(comm, ssems, rsems, dev, 3, 0, sp[0], pp[0], 0)

                @pl.when(s >= 2)
                def _():
                    d.wait_send()
                comm[sp[0], 3, :, :] = part(chunks[0], w0_ref, lo).astype(
                    jnp.bfloat16
                )
                d.start()

        @pl.when(jnp.logical_and(s >= 1, s <= T))
        def _():
            for comm, ssems, rsems, dev, lo, chunks in dirs:
                prv = part(chunks[1], w1_ref, lo)
                rec = fwd(comm, ssems, rsems, dev, 0, 0, sp[1], pp[1], 0)
                rec.wait_recv()
                snd = fwd(comm, ssems, rsems, dev, 0, 1, sp[1], pp[1], 1)

                @pl.when(s >= 3)
                def _():
                    snd.wait_send()
                comm[sp[1], 0, :, :] = (
                    comm[sp[1], 0, :, :].astype(jnp.float32) + prv
                ).astype(jnp.bfloat16)
                snd.start()

        @pl.when(jnp.logical_and(s >= 2, s <= T + 1))
        def _():
            for comm, ssems, rsems, dev, lo, chunks in dirs:
                prv = part(chunks[2], w2_ref, lo)
                rec = fwd(comm, ssems, rsems, dev, 1, 1, sp[2], pp[2], 1)
                rec.wait_recv()
                snd = fwd(comm, ssems, rsems, dev, 1, 2, sp[2], pp[2], 2)

                @pl.when(s >= 4)
                def _():
                    snd.wait_send()
                comm[sp[2], 1, :, :] = (
                    comm[sp[2], 1, :, :].astype(jnp.float32) + prv
                ).astype(jnp.bfloat16)
                snd.start()

        @pl.when(s >= 3)
        def _():
            for comm, ssems, rsems, dev, lo, chunks in dirs:
                prv = part(chunks[3], w3_ref, lo)
                rec = fwd(comm, ssems, rsems, dev, 2, 2, sp[3], pp[3], 2)
                rec.wait_recv()
                out_ref[:, lo:lo + H] = (
                    comm[sp[3], 2, :, :].astype(jnp.float32) + prv
                )

        pl.semaphore_signal(
            credit_r, inc=1,
            device_id=(left,), device_id_type=pl.DeviceIdType.MESH,
        )
        pl.semaphore_signal(
            credit_l, inc=1,
            device_id=(right,), device_id_type=pl.DeviceIdType.MESH,
        )

        @pl.when(s == S - 1)
        def _():
            for comm, ssems, rsems, dev, lo, chunks in dirs:
                for par in (0, 1):
                    for h in range(3):
                        fwd(comm, ssems, rsems, dev, 3, 0, 0, par, h).wait_send()
            pl.semaphore_wait(credit_r, 1)
            pl.semaphore_wait(credit_l, 1)

    clamp = lambda v: jnp.clip(v, 0, T - 1)
    return pl.pallas_call(
        body,
        grid=(S,),
        out_shape=jax.ShapeDtypeStruct((m_blk, n_total), jnp.float32),
        in_specs=[
            pl.BlockSpec(memory_space=pltpu.VMEM),
            pl.BlockSpec((k_loc, NT), lambda s: (0, clamp(s))),
            pl.BlockSpec((k_loc, NT), lambda s: (0, clamp(s - 1))),
            pl.BlockSpec((k_loc, NT), lambda s: (0, clamp(s - 2))),
            pl.BlockSpec((k_loc, NT), lambda s: (0, clamp(s - 3))),
        ],
        out_specs=pl.BlockSpec((m_blk, NT), lambda s: (0, clamp(s - 3))),
        scratch_shapes=[
            pltpu.VMEM((4, 4, m_blk, H), jnp.bfloat16),
            pltpu.VMEM((4, 4, m_blk, H), jnp.bfloat16),
            pltpu.SemaphoreType.DMA((2, 3)),
            pltpu.SemaphoreType.DMA((2, 3)),
            pltpu.SemaphoreType.DMA((2, 3)),
            pltpu.SemaphoreType.DMA((2, 3)),
            pltpu.SemaphoreType.REGULAR,
            pltpu.SemaphoreType.REGULAR,
        ],
        compiler_params=pltpu.CompilerParams(
            collective_id=0,
            vmem_limit_bytes=64 * 1024 * 1024,
        ),
    )(x, w_mat, w_mat, w_mat, w_mat)


# device time: 311113 ns/iter; 1.0756x vs baseline; 1.0756x over previous
import jax
import jax.numpy as jnp
from jax import lax
from jax.experimental import pallas as pl
from jax.experimental.pallas import tpu as pltpu

N_DEV = 4
NT = 512
H = NT // 2


def kernel(x, w_mat):
    m_total, k_loc = x.shape
    k_loc2, n_total = w_mat.shape
    assert k_loc == k_loc2
    m_blk = m_total // N_DEV
    T = n_total // NT
    S = T + 3

    def body(x_ref, w0_ref, w1_ref, w2_ref, w3_ref, out_ref,
             comm_r, comm_l, send_r, recv_r, send_l, recv_l,
             credit_r, credit_l):
        s = pl.program_id(0)
        my = lax.axis_index("i")
        left = (my + N_DEV - 1) % N_DEV
        right = (my + 1) % N_DEV

        @pl.when(s == 0)
        def _():
            barrier_sem = pltpu.get_barrier_semaphore()
            for nbr in (left, right):
                pl.semaphore_signal(
                    barrier_sem, inc=1,
                    device_id=(nbr,), device_id_type=pl.DeviceIdType.MESH,
                )
            pl.semaphore_wait(barrier_sem, 2)

        @pl.when(s >= 1)
        def _():
            pl.semaphore_wait(credit_r, 1)
            pl.semaphore_wait(credit_l, 1)

        sp = [(s + 4 - k) % 4 for k in range(4)]
        pp = [(s + 2 - k) % 2 for k in range(4)]

        def part(j, w_ref, lo):
            xb = x_ref[pl.ds(j * m_blk, m_blk), :]
            return jnp.dot(
                xb, w_ref[:, lo:lo + H], preferred_element_type=jnp.float32
            )

        dirs = (
            (comm_r, send_r, recv_r, right, 0,
             ((my + 3) % N_DEV, (my + 2) % N_DEV, (my + 1) % N_DEV, my)),
            (comm_l, send_l, recv_l, left, H,
             ((my + 1) % N_DEV, (my + 2) % N_DEV, (my + 3) % N_DEV, my)),
        )

        def fwd(comm, ssems, rsems, dev, src, dst, spar, ppar, h):
            return pltpu.make_async_remote_copy(
                src_ref=comm.at[spar, src],
                dst_ref=comm.at[spar, dst],
                send_sem=ssems.at[ppar, h],
                recv_sem=rsems.at[ppar, h],
                device_id=(dev,),
                device_id_type=pl.DeviceIdType.MESH,
            )

        @pl.when(s < T)
        def _():
            for comm, ssems, rsems, dev, lo, chunks in dirs:
                d = fwd(comm, ssems, rsems, dev, 3, 0, sp[0], pp[0], 0)

                @pl.when(s >= 2)
                def _():
                    d.wait_send()
                comm[sp[0], 3, :, :] = part(chunks[0], w0_ref, lo).astype(
                    jnp.bfloat16
                )
                d.start()

        @pl.when(jnp.logical_and(s >= 1, s <= T))
        def _():
            for comm, ssems, rsems, dev, lo, chunks in dirs:
                prv = part(chunks[1], w1_ref, lo)
                rec = fwd(comm, ssems, rsems, dev, 0, 0, sp[1], pp[1], 0)
                rec.wait_recv()
                snd = fwd(comm, ssems, rsems, dev, 0, 1, sp[1], pp[1], 1)

                @pl.when(s >= 3)
                def _():
                    snd.wait_send()
                comm[sp[1], 0, :, :] = (
                    comm[sp[1], 0, :, :].astype(jnp.float32) + prv
                ).astype(jnp.bfloat16)
                snd.start()

        @pl.when(jnp.logical_and(s >= 2, s <= T + 1))
        def _():
            for comm, ssems, rsems, dev, lo, chunks in dirs:
                prv = part(chunks[2], w2_ref, lo)
                rec = fwd(comm, ssems, rsems, dev, 1, 1, sp[2], pp[2], 1)
                rec.wait_recv()
                snd = fwd(comm, ssems, rsems, dev, 1, 2, sp[2], pp[2], 2)

                @pl.when(s >= 4)
                def _():
                    snd.wait_send()
                comm[sp[2], 1, :, :] = (
                    comm[sp[2], 1, :, :].astype(jnp.float32) + prv
                ).astype(jnp.bfloat16)
                snd.start()

        @pl.when(s >= 3)
        def _():
            for comm, ssems, rsems, dev, lo, chunks in dirs:
                prv = part(chunks[3], w3_ref, lo)
                rec = fwd(comm, ssems, rsems, dev, 2, 2, sp[3], pp[3], 2)
                rec.wait_recv()
                out_ref[:, lo:lo + H] = (
                    comm[sp[3], 2, :, :].astype(jnp.float32) + prv
                )

        pl.semaphore_signal(
            credit_r, inc=1,
            device_id=(left,), device_id_type=pl.DeviceIdType.MESH,
        )
        pl.semaphore_signal(
            credit_l, inc=1,
            device_id=(right,), device_id_type=pl.DeviceIdType.MESH,
        )

        @pl.when(s == S - 1)
        def _():
            for comm, ssems, rsems, dev, lo, chunks in dirs:
                for par in (0, 1):
                    for h in range(3):
                        fwd(comm, ssems, rsems, dev, 3, 0, 0, par, h).wait_send()
            pl.semaphore_wait(credit_r, 1)
            pl.semaphore_wait(credit_l, 1)

    clamp = lambda v: jnp.clip(v, 0, T - 1)
    return pl.pallas_call(
        body,
        grid=(S,),
        out_shape=jax.ShapeDtypeStruct((m_blk, n_total), jnp.float32),
        in_specs=[
            pl.BlockSpec(memory_space=pltpu.VMEM),
            pl.BlockSpec((k_loc, NT), lambda s: (0, clamp(s))),
            pl.BlockSpec((k_loc, NT), lambda s: (0, clamp(s - 1))),
            pl.BlockSpec((k_loc, NT), lambda s: (0, clamp(s - 2))),
            pl.BlockSpec((k_loc, NT), lambda s: (0, clamp(s - 3))),
        ],
        out_specs=pl.BlockSpec((m_blk, NT), lambda s: (0, clamp(s - 3))),
        scratch_shapes=[
            pltpu.VMEM((4, 4, m_blk, H), jnp.bfloat16),
            pltpu.VMEM((4, 4, m_blk, H), jnp.bfloat16),
            pltpu.SemaphoreType.DMA((2, 3)),
            pltpu.SemaphoreType.DMA((2, 3)),
            pltpu.SemaphoreType.DMA((2, 3)),
            pltpu.SemaphoreType.DMA((2, 3)),
            pltpu.SemaphoreType.REGULAR,
            pltpu.SemaphoreType.REGULAR,
        ],
        compiler_params=pltpu.CompilerParams(
            collective_id=0,
            vmem_limit_bytes=64 * 1024 * 1024,
        ),
    )(x, w_mat, w_mat, w_mat, w_mat)
